# baseline (speedup 1.0000x reference)
import jax
import jax.numpy as jnp
from jax.experimental import pallas as pl
from jax.experimental.pallas import tpu as pltpu

_M = 256
_B = 128


def _route_and_copy(idx_ref,
                    m0, m1, m2, m3, m4,
                    v0, v1, v2, v3, v4,
                    o0, o1, o2, o3, o4,
                    route_ref, s0, s1, s2, s3, s4):
    mems = (m0, m1, m2, m3, m4)
    vals = (v0, v1, v2, v3, v4)
    outs = (o0, o1, o2, o3, o4)
    sems = (s0, s1, s2, s3, s4)

    def _init(m, c):
        route_ref[m] = -1
        return c
    jax.lax.fori_loop(0, _M, _init, 0)

    def _setr(j, c):
        route_ref[idx_ref[j]] = j
        return c
    jax.lax.fori_loop(0, _B, _setr, 0)

    def _per_slot(m, c):
        j = route_ref[m]

        @pl.when(j >= 0)
        def _():
            jj = jnp.maximum(j, 0)
            for val, out, sem in zip(vals, outs, sems):
                pltpu.make_async_copy(val.at[jj], out.at[m], sem).start()

        @pl.when(j < 0)
        def _():
            for mem, out, sem in zip(mems, outs, sems):
                pltpu.make_async_copy(mem.at[m], out.at[m], sem).start()
        return c
    jax.lax.fori_loop(0, _M, _per_slot, 0)

    def _drain(m, c):
        for mem, out, sem in zip(mems, outs, sems):
            pltpu.make_async_copy(mem.at[0], out.at[0], sem).wait()
        return c
    jax.lax.fori_loop(0, _M, _drain, 0)


def kernel(x_i_mem, y_j_mem, x_i_new_mem, y_j_new_mem, P_mem,
           x_i_val, y_j_val, x_i_new_val, y_j_new_val, P_val, idx):
    mems = [x_i_mem.reshape(_M, -1), y_j_mem.reshape(_M, -1),
            x_i_new_mem.reshape(_M, -1), y_j_new_mem.reshape(_M, -1),
            P_mem.reshape(_M, -1)]
    vals = [x_i_val.reshape(_B, -1), y_j_val.reshape(_B, -1),
            x_i_new_val.reshape(_B, -1), y_j_new_val.reshape(_B, -1),
            P_val.reshape(_B, -1)]

    out2d = pl.pallas_call(
        _route_and_copy,
        in_specs=[pl.BlockSpec(memory_space=pltpu.SMEM)]
                 + [pl.BlockSpec(memory_space=pl.ANY)] * 10,
        out_specs=[pl.BlockSpec(memory_space=pl.ANY)] * 5,
        out_shape=[jax.ShapeDtypeStruct(m.shape, m.dtype) for m in mems],
        scratch_shapes=[pltpu.SMEM((_M,), jnp.int32)]
                       + [pltpu.SemaphoreType.DMA] * 5,
    )(idx, *mems, *vals)

    shapes = [x_i_mem.shape, y_j_mem.shape, x_i_new_mem.shape,
              y_j_new_mem.shape, P_mem.shape]
    return tuple(o.reshape(s) for o, s in zip(out2d, shapes))


# speedup vs baseline: 7.6161x; 7.6161x over previous
import jax
import jax.numpy as jnp
from jax.experimental import pallas as pl
from jax.experimental.pallas import tpu as pltpu

_M = 256
_B = 128


def _route(idx_ref, fl_ref, ms_ref, vs_ref):
    def _init(m, c):
        fl_ref[m] = 0
        ms_ref[m] = m
        vs_ref[m] = -1
        return c
    jax.lax.fori_loop(0, _M, _init, 0)

    def _setv(j, c):
        fl_ref[idx_ref[j]] = 1
        vs_ref[idx_ref[j]] = j
        return c
    jax.lax.fori_loop(0, _B, _setv, 0)

    def _fill(m, c):
        upd = fl_ref[m] == 1
        prev = jnp.maximum(m - 1, 0)
        ms_ref[m] = jnp.where(upd, ms_ref[prev], m)
        vs_ref[m] = jnp.where(upd, vs_ref[m], jnp.maximum(vs_ref[prev], 0))
        return c
    jax.lax.fori_loop(0, _M, _fill, 0)


def _select(fl_ref, ms_ref, vs_ref,
            m0, m1, m2, m3, m4, v0, v1, v2, v3, v4,
            o0, o1, o2, o3, o4):
    m = pl.program_id(0)
    upd = fl_ref[m] == 1

    @pl.when(upd)
    def _():
        for v, o in zip((v0, v1, v2, v3, v4), (o0, o1, o2, o3, o4)):
            o[...] = v[...]

    @pl.when(jnp.logical_not(upd))
    def _():
        for mm, o in zip((m0, m1, m2, m3, m4), (o0, o1, o2, o3, o4)):
            o[...] = mm[...]


def kernel(x_i_mem, y_j_mem, x_i_new_mem, y_j_new_mem, P_mem,
           x_i_val, y_j_val, x_i_new_val, y_j_new_val, P_val, idx):
    mems = (x_i_mem, y_j_mem, x_i_new_mem, y_j_new_mem,
            P_mem.reshape(_M, 20, 1000))
    vals = (x_i_val, y_j_val, x_i_new_val, y_j_new_val,
            P_val.reshape(_B, 20, 1000))

    fl, ms, vs = pl.pallas_call(
        _route,
        in_specs=[pl.BlockSpec(memory_space=pltpu.SMEM)],
        out_specs=[pl.BlockSpec(memory_space=pltpu.SMEM)] * 3,
        out_shape=[jax.ShapeDtypeStruct((_M,), jnp.int32)] * 3,
    )(idx)

    def _mem_spec(shape):
        rest = shape[1:]
        zeros = (0,) * len(rest)
        return pl.BlockSpec((1,) + rest,
                            lambda m, fl, ms, vs, z=zeros: (ms[m],) + z)

    def _val_spec(shape):
        rest = shape[1:]
        zeros = (0,) * len(rest)
        return pl.BlockSpec((1,) + rest,
                            lambda m, fl, ms, vs, z=zeros: (vs[m],) + z)

    def _out_spec(shape):
        rest = shape[1:]
        zeros = (0,) * len(rest)
        return pl.BlockSpec((1,) + rest,
                            lambda m, fl, ms, vs, z=zeros: (m,) + z)

    grid_spec = pltpu.PrefetchScalarGridSpec(
        num_scalar_prefetch=3,
        grid=(_M,),
        in_specs=[_mem_spec(t.shape) for t in mems]
                 + [_val_spec(t.shape) for t in vals],
        out_specs=[_out_spec(t.shape) for t in mems],
    )

    outs = pl.pallas_call(
        _select,
        grid_spec=grid_spec,
        out_shape=[jax.ShapeDtypeStruct(t.shape, t.dtype) for t in mems],
    )(fl, ms, vs, *mems, *vals)
    return (outs[0], outs[1], outs[2], outs[3], outs[4].reshape(P_mem.shape))
